# baseline (speedup 1.0000x reference)
import functools

import jax
import jax.numpy as jnp
from jax import lax
from jax.experimental import pallas as pl
from jax.experimental.pallas import tpu as pltpu
from jax.experimental.pallas import tpu_sc as plsc

BATCH = 16384
EMBED_DIM = 64
ITEM_OFFSET = 1000000

NC = 2
NS = 16
NW = NC * NS
BPW = BATCH // NW
NCHUNK = 4
CHUNK = BPW // NCHUNK


def _sc_body(x_hbm, table_hbm, out_hbm, xv, idxu, idxi, rows_u, rows_i,
             out_v, sem):
    wid = lax.axis_index("s") * NC + lax.axis_index("c")
    base = wid * BPW

    pltpu.sync_copy(x_hbm.at[pl.ds(base, BPW)], xv)

    iota16 = lax.iota(jnp.int32, 16)
    col0 = jnp.zeros((16,), jnp.int32)
    col1 = jnp.ones((16,), jnp.int32)

    for g in range(BPW // 16):
        rid = g * 16 + iota16
        u = plsc.load_gather(xv, [rid, col0])
        it = plsc.load_gather(xv, [rid, col1]) + ITEM_OFFSET
        j, off = divmod(g * 16, CHUNK)
        idxu[j, pl.ds(off, 16)] = u
        idxi[j, pl.ds(off, 16)] = it

    copies = []
    for j in range(NCHUNK):
        copies.append(pltpu.async_copy(
            table_hbm.at[idxu.at[j]], rows_u.at[pl.ds(j * CHUNK, CHUNK)], sem))
        copies.append(pltpu.async_copy(
            table_hbm.at[idxi.at[j]], rows_i.at[pl.ds(j * CHUNK, CHUNK)], sem))
    for cp in copies:
        cp.wait()

    def dot_group(g, carry):
        rid = g * 16 + iota16
        acc = jnp.zeros((16,), jnp.float32)
        for d in range(EMBED_DIM):
            cd = jnp.full((16,), d, jnp.int32)
            uu = plsc.load_gather(rows_u, [rid, cd])
            vv = plsc.load_gather(rows_i, [rid, cd])
            acc = acc + uu * vv
        out_v[pl.ds(g * 16, 16)] = acc
        return carry

    lax.fori_loop(0, BPW // 16, dot_group, 0)

    pltpu.sync_copy(out_v, out_hbm.at[pl.ds(base, BPW)])


def _sc_dot(x, table):
    mesh = plsc.VectorSubcoreMesh(core_axis_name="c", subcore_axis_name="s")
    kern = functools.partial(
        pl.kernel,
        out_type=jax.ShapeDtypeStruct((BATCH,), jnp.float32),
        mesh=mesh,
        compiler_params=pltpu.CompilerParams(needs_layout_passes=False,
                                             use_tc_tiling_on_sc=False),
        scratch_types=[
            pltpu.VMEM((BPW, 2), jnp.int32),
            pltpu.VMEM((NCHUNK, CHUNK), jnp.int32),
            pltpu.VMEM((NCHUNK, CHUNK), jnp.int32),
            pltpu.VMEM((BPW, EMBED_DIM), jnp.float32),
            pltpu.VMEM((BPW, EMBED_DIM), jnp.float32),
            pltpu.VMEM((BPW,), jnp.float32),
            pltpu.SemaphoreType.DMA,
        ],
    )(_sc_body)
    return kern(x, table)


def kernel(x, table):
    y = _sc_dot(x.astype(jnp.int32), table)
    return y.reshape(BATCH, 1)


# speedup vs baseline: 11.9352x; 11.9352x over previous
import functools

import jax
import jax.numpy as jnp
from jax import lax
from jax.experimental import pallas as pl
from jax.experimental.pallas import tpu as pltpu
from jax.experimental.pallas import tpu_sc as plsc

BATCH = 16384
EMBED_DIM = 64
ROWS = 2000000
ITEM_OFFSET = 1000000

NC = 2
NS = 16
NW = NC * NS
BPW = BATCH // NW
NHALF = 2
HALF = BPW // NHALF
NG = HALF // 16
IDX_PER_HALF = HALF * EMBED_DIM
CH = 2048

A_STRIDE = (ROWS // 128) * 8 * 128
B_STRIDE = 8 * 128
C_STRIDE = 128


def _sc_body(xu_hbm, xi_hbm, tflat_hbm, out_hbm,
             xu_v, xi_v, idx_u, idx_i, dst_u, dst_i, out_v, sem):
    wid = lax.axis_index("s") * NC + lax.axis_index("c")
    base = wid * BPW

    pltpu.sync_copy(xu_hbm.at[pl.ds(base, BPW)], xu_v)
    pltpu.sync_copy(xi_hbm.at[pl.ds(base, BPW)], xi_v)

    for half in range(NHALF):
        h0 = half * HALF

        def gen(g, carry):
            u = xu_v[pl.ds(h0 + g * 16, 16)]
            it = xi_v[pl.ds(h0 + g * 16, 16)] + ITEM_OFFSET
            bu = lax.shift_right_logical(u, 7) * B_STRIDE + (u & 127)
            bi = lax.shift_right_logical(it, 7) * B_STRIDE + (it & 127)
            for a in range(8):
                for c in range(8):
                    off = a * A_STRIDE + c * C_STRIDE
                    p = g * 1024 + (a * 8 + c) * 16
                    idx_u[pl.ds(p, 16)] = bu + off
                    idx_i[pl.ds(p, 16)] = bi + off
            return carry

        lax.fori_loop(0, NG, gen, 0)

        copies = []
        for j in range(IDX_PER_HALF // CH):
            copies.append(pltpu.async_copy(
                tflat_hbm.at[idx_u.at[pl.ds(j * CH, CH)]],
                dst_u.at[pl.ds(j * CH, CH)], sem))
            copies.append(pltpu.async_copy(
                tflat_hbm.at[idx_i.at[pl.ds(j * CH, CH)]],
                dst_i.at[pl.ds(j * CH, CH)], sem))
        for cp in copies:
            cp.wait()

        def dot(g, carry):
            acc = jnp.zeros((16,), jnp.float32)
            for k in range(EMBED_DIM):
                p = g * 1024 + k * 16
                acc = acc + dst_u[pl.ds(p, 16)] * dst_i[pl.ds(p, 16)]
            out_v[pl.ds(h0 + g * 16, 16)] = acc
            return carry

        lax.fori_loop(0, NG, dot, 0)

    pltpu.sync_copy(out_v, out_hbm.at[pl.ds(base, BPW)])


def _sc_dot(xu, xi, tflat):
    mesh = plsc.VectorSubcoreMesh(core_axis_name="c", subcore_axis_name="s")
    kern = functools.partial(
        pl.kernel,
        out_type=jax.ShapeDtypeStruct((BATCH,), jnp.float32),
        mesh=mesh,
        compiler_params=pltpu.CompilerParams(needs_layout_passes=False,
                                             use_tc_tiling_on_sc=False),
        scratch_types=[
            pltpu.VMEM((BPW,), jnp.int32),
            pltpu.VMEM((BPW,), jnp.int32),
            pltpu.VMEM((IDX_PER_HALF,), jnp.int32),
            pltpu.VMEM((IDX_PER_HALF,), jnp.int32),
            pltpu.VMEM((IDX_PER_HALF,), jnp.float32),
            pltpu.VMEM((IDX_PER_HALF,), jnp.float32),
            pltpu.VMEM((BPW,), jnp.float32),
            pltpu.SemaphoreType.DMA,
        ],
    )(_sc_body)
    return kern(xu, xi, tflat)


def kernel(x, table):
    x = x.astype(jnp.int32)
    tflat = (table.T.reshape(8, 8, ROWS // 128, 128)
             .transpose(0, 2, 1, 3).reshape(-1))
    y = _sc_dot(x[:, 0], x[:, 1], tflat)
    return y.reshape(BATCH, 1)


# speedup vs baseline: 11.9706x; 1.0030x over previous
import functools

import jax
import jax.numpy as jnp
from jax import lax
from jax.experimental import pallas as pl
from jax.experimental.pallas import tpu as pltpu
from jax.experimental.pallas import tpu_sc as plsc

BATCH = 16384
EMBED_DIM = 64
ROWS = 2000000
ITEM_OFFSET = 1000000

NC = 2
NS = 16
NW = NC * NS
BPW = BATCH // NW
NQ = 4
QROWS = BPW // NQ
QG = QROWS // 16
IDX_PER_Q = QROWS * EMBED_DIM
CH = 2048

A_STRIDE = (ROWS // 128) * 8 * 128
B_STRIDE = 8 * 128
C_STRIDE = 128


def _sc_body(xu_hbm, xi_hbm, tflat_hbm, out_hbm,
             xu_v, xi_v, idx_u0, idx_i0, idx_u1, idx_i1,
             dst_u0, dst_i0, dst_u1, dst_i1, out_v, sem0, sem1):
    wid = lax.axis_index("s") * NC + lax.axis_index("c")
    base = wid * BPW

    pltpu.sync_copy(xu_hbm.at[pl.ds(base, BPW)], xu_v)
    pltpu.sync_copy(xi_hbm.at[pl.ds(base, BPW)], xi_v)

    bufs = [(idx_u0, idx_i0, dst_u0, dst_i0, sem0),
            (idx_u1, idx_i1, dst_u1, dst_i1, sem1)]

    def gen_q(q, idx_u, idx_i):
        h0 = q * QROWS

        def gen(g, carry):
            u = xu_v[pl.ds(h0 + g * 16, 16)]
            it = xi_v[pl.ds(h0 + g * 16, 16)] + ITEM_OFFSET
            bu = lax.shift_right_logical(u, 7) * B_STRIDE + (u & 127)
            bi = lax.shift_right_logical(it, 7) * B_STRIDE + (it & 127)
            for a in range(8):
                for c in range(8):
                    off = a * A_STRIDE + c * C_STRIDE
                    p = g * 1024 + (a * 8 + c) * 16
                    idx_u[pl.ds(p, 16)] = bu + off
                    idx_i[pl.ds(p, 16)] = bi + off
            return carry

        lax.fori_loop(0, QG, gen, 0)

    def fire_q(idx_u, idx_i, dst_u, dst_i, sem):
        copies = []
        for j in range(IDX_PER_Q // CH):
            copies.append(pltpu.async_copy(
                tflat_hbm.at[idx_u.at[pl.ds(j * CH, CH)]],
                dst_u.at[pl.ds(j * CH, CH)], sem))
            copies.append(pltpu.async_copy(
                tflat_hbm.at[idx_i.at[pl.ds(j * CH, CH)]],
                dst_i.at[pl.ds(j * CH, CH)], sem))
        return copies

    def dot_q(q, dst_u, dst_i):
        h0 = q * QROWS

        def dot(g, carry):
            acc = jnp.zeros((16,), jnp.float32)
            for k in range(EMBED_DIM):
                p = g * 1024 + k * 16
                acc = acc + dst_u[pl.ds(p, 16)] * dst_i[pl.ds(p, 16)]
            out_v[pl.ds(h0 + g * 16, 16)] = acc
            return carry

        lax.fori_loop(0, QG, dot, 0)

    inflight = {}
    for q in range(NQ):
        slot = q % 2
        if q >= 2:
            for cp in inflight[q - 2]:
                cp.wait()
            dot_q(q - 2, bufs[slot][2], bufs[slot][3])
        gen_q(q, bufs[slot][0], bufs[slot][1])
        inflight[q] = fire_q(*bufs[slot])
    for q in (NQ - 2, NQ - 1):
        slot = q % 2
        for cp in inflight[q]:
            cp.wait()
        dot_q(q, bufs[slot][2], bufs[slot][3])

    pltpu.sync_copy(out_v, out_hbm.at[pl.ds(base, BPW)])


def _sc_dot(xu, xi, tflat):
    mesh = plsc.VectorSubcoreMesh(core_axis_name="c", subcore_axis_name="s")
    kern = functools.partial(
        pl.kernel,
        out_type=jax.ShapeDtypeStruct((BATCH,), jnp.float32),
        mesh=mesh,
        compiler_params=pltpu.CompilerParams(needs_layout_passes=False,
                                             use_tc_tiling_on_sc=False),
        scratch_types=[
            pltpu.VMEM((BPW,), jnp.int32),
            pltpu.VMEM((BPW,), jnp.int32),
            pltpu.VMEM((IDX_PER_Q,), jnp.int32),
            pltpu.VMEM((IDX_PER_Q,), jnp.int32),
            pltpu.VMEM((IDX_PER_Q,), jnp.int32),
            pltpu.VMEM((IDX_PER_Q,), jnp.int32),
            pltpu.VMEM((IDX_PER_Q,), jnp.float32),
            pltpu.VMEM((IDX_PER_Q,), jnp.float32),
            pltpu.VMEM((IDX_PER_Q,), jnp.float32),
            pltpu.VMEM((IDX_PER_Q,), jnp.float32),
            pltpu.VMEM((BPW,), jnp.float32),
            pltpu.SemaphoreType.DMA,
            pltpu.SemaphoreType.DMA,
        ],
    )(_sc_body)
    return kern(xu, xi, tflat)


def kernel(x, table):
    x = x.astype(jnp.int32)
    tflat = (table.T.reshape(8, 8, ROWS // 128, 128)
             .transpose(0, 2, 1, 3).reshape(-1))
    y = _sc_dot(x[:, 0], x[:, 1], tflat)
    return y.reshape(BATCH, 1)


# speedup vs baseline: 12.0448x; 1.0062x over previous
import functools

import jax
import jax.numpy as jnp
from jax import lax
from jax.experimental import pallas as pl
from jax.experimental.pallas import tpu as pltpu
from jax.experimental.pallas import tpu_sc as plsc

BATCH = 16384
EMBED_DIM = 64
ROWS = 2000000
ITEM_OFFSET = 1000000

NC = 2
NS = 16
NW = NC * NS
BPW = BATCH // NW
NQ = 4
QROWS = BPW // NQ
QG = QROWS // 16
IDX_PER_Q = QROWS * EMBED_DIM
CH = 8192

A_STRIDE = (ROWS // 128) * 8 * 128
B_STRIDE = 8 * 128
C_STRIDE = 128


def _sc_body(xu_hbm, xi_hbm, tflat_hbm, out_hbm,
             xu_v, xi_v, idx_u0, idx_i0, idx_u1, idx_i1,
             dst_u0, dst_i0, dst_u1, dst_i1, out_v, sem0, sem1):
    wid = lax.axis_index("s") * NC + lax.axis_index("c")
    base = wid * BPW

    pltpu.sync_copy(xu_hbm.at[pl.ds(base, BPW)], xu_v)
    pltpu.sync_copy(xi_hbm.at[pl.ds(base, BPW)], xi_v)

    bufs = [(idx_u0, idx_i0, dst_u0, dst_i0, sem0),
            (idx_u1, idx_i1, dst_u1, dst_i1, sem1)]

    def gen_q(q, idx_u, idx_i):
        h0 = q * QROWS

        def gen(g, carry):
            u = xu_v[pl.ds(h0 + g * 16, 16)]
            it = xi_v[pl.ds(h0 + g * 16, 16)] + ITEM_OFFSET
            bu = lax.shift_right_logical(u, 7) * B_STRIDE + (u & 127)
            bi = lax.shift_right_logical(it, 7) * B_STRIDE + (it & 127)
            for a in range(8):
                for c in range(8):
                    off = a * A_STRIDE + c * C_STRIDE
                    p = g * 1024 + (a * 8 + c) * 16
                    idx_u[pl.ds(p, 16)] = bu + off
                    idx_i[pl.ds(p, 16)] = bi + off
            return carry

        lax.fori_loop(0, QG, gen, 0)

    def fire_q(idx_u, idx_i, dst_u, dst_i, sem):
        copies = []
        for j in range(IDX_PER_Q // CH):
            copies.append(pltpu.async_copy(
                tflat_hbm.at[idx_u.at[pl.ds(j * CH, CH)]],
                dst_u.at[pl.ds(j * CH, CH)], sem))
            copies.append(pltpu.async_copy(
                tflat_hbm.at[idx_i.at[pl.ds(j * CH, CH)]],
                dst_i.at[pl.ds(j * CH, CH)], sem))
        return copies

    def dot_q(q, dst_u, dst_i):
        h0 = q * QROWS

        def dot(g, carry):
            acc = jnp.zeros((16,), jnp.float32)
            for k in range(EMBED_DIM):
                p = g * 1024 + k * 16
                acc = acc + dst_u[pl.ds(p, 16)] * dst_i[pl.ds(p, 16)]
            out_v[pl.ds(h0 + g * 16, 16)] = acc
            return carry

        lax.fori_loop(0, QG, dot, 0)

    inflight = {}
    for q in range(NQ):
        slot = q % 2
        if q >= 2:
            for cp in inflight[q - 2]:
                cp.wait()
            dot_q(q - 2, bufs[slot][2], bufs[slot][3])
        gen_q(q, bufs[slot][0], bufs[slot][1])
        inflight[q] = fire_q(*bufs[slot])
    for q in (NQ - 2, NQ - 1):
        slot = q % 2
        for cp in inflight[q]:
            cp.wait()
        dot_q(q, bufs[slot][2], bufs[slot][3])

    pltpu.sync_copy(out_v, out_hbm.at[pl.ds(base, BPW)])


def _sc_dot(xu, xi, tflat):
    mesh = plsc.VectorSubcoreMesh(core_axis_name="c", subcore_axis_name="s")
    kern = functools.partial(
        pl.kernel,
        out_type=jax.ShapeDtypeStruct((BATCH,), jnp.float32),
        mesh=mesh,
        compiler_params=pltpu.CompilerParams(needs_layout_passes=False,
                                             use_tc_tiling_on_sc=False),
        scratch_types=[
            pltpu.VMEM((BPW,), jnp.int32),
            pltpu.VMEM((BPW,), jnp.int32),
            pltpu.VMEM((IDX_PER_Q,), jnp.int32),
            pltpu.VMEM((IDX_PER_Q,), jnp.int32),
            pltpu.VMEM((IDX_PER_Q,), jnp.int32),
            pltpu.VMEM((IDX_PER_Q,), jnp.int32),
            pltpu.VMEM((IDX_PER_Q,), jnp.float32),
            pltpu.VMEM((IDX_PER_Q,), jnp.float32),
            pltpu.VMEM((IDX_PER_Q,), jnp.float32),
            pltpu.VMEM((IDX_PER_Q,), jnp.float32),
            pltpu.VMEM((BPW,), jnp.float32),
            pltpu.SemaphoreType.DMA,
            pltpu.SemaphoreType.DMA,
        ],
    )(_sc_body)
    return kern(xu, xi, tflat)


def kernel(x, table):
    x = x.astype(jnp.int32)
    tflat = (table.T.reshape(8, 8, ROWS // 128, 128)
             .transpose(0, 2, 1, 3).reshape(-1))
    y = _sc_dot(x[:, 0], x[:, 1], tflat)
    return y.reshape(BATCH, 1)


# speedup vs baseline: 12.1539x; 1.0091x over previous
import functools

import jax
import jax.numpy as jnp
from jax import lax
from jax.experimental import pallas as pl
from jax.experimental.pallas import tpu as pltpu
from jax.experimental.pallas import tpu_sc as plsc

BATCH = 16384
EMBED_DIM = 64
ROWS = 2000000
ITEM_OFFSET = 1000000

NC = 2
NS = 16
NW = NC * NS
BPW = BATCH // NW
NQ = 4
QROWS = BPW // NQ
QG = QROWS // 16
IDX_PER_Q = QROWS * EMBED_DIM
CH = 8192

A_STRIDE = (ROWS // 128) * 8 * 128
B_STRIDE = 8 * 128
C_STRIDE = 128


def _sc_body(xu_hbm, xi_hbm, tflat_hbm, out_hbm,
             xu_v, xi_v, idx_u0, idx_i0, idx_u1, idx_i1, idx_u2, idx_i2,
             dst_u0, dst_i0, dst_u1, dst_i1, dst_u2, dst_i2,
             out_v, sem0, sem1, sem2, semx):
    wid = lax.axis_index("s") * NC + lax.axis_index("c")
    base = wid * BPW

    cpu = pltpu.async_copy(xu_hbm.at[pl.ds(base, BPW)], xu_v, semx)
    cpi = pltpu.async_copy(xi_hbm.at[pl.ds(base, BPW)], xi_v, semx)
    cpu.wait()
    cpi.wait()

    bufs = [(idx_u0, idx_i0, dst_u0, dst_i0, sem0),
            (idx_u1, idx_i1, dst_u1, dst_i1, sem1),
            (idx_u2, idx_i2, dst_u2, dst_i2, sem2)]

    def gen_q(q, idx_u, idx_i):
        h0 = q * QROWS

        def gen(g, carry):
            u = xu_v[pl.ds(h0 + g * 16, 16)]
            it = xi_v[pl.ds(h0 + g * 16, 16)] + ITEM_OFFSET
            bu = lax.shift_right_logical(u, 7) * B_STRIDE + (u & 127)
            bi = lax.shift_right_logical(it, 7) * B_STRIDE + (it & 127)
            for a in range(8):
                for c in range(8):
                    off = a * A_STRIDE + c * C_STRIDE
                    p = g * 1024 + (a * 8 + c) * 16
                    idx_u[pl.ds(p, 16)] = bu + off
                    idx_i[pl.ds(p, 16)] = bi + off
            return carry

        lax.fori_loop(0, QG, gen, 0)

    def fire_q(idx_u, idx_i, dst_u, dst_i, sem):
        copies = []
        for j in range(IDX_PER_Q // CH):
            copies.append(pltpu.async_copy(
                tflat_hbm.at[idx_u.at[pl.ds(j * CH, CH)]],
                dst_u.at[pl.ds(j * CH, CH)], sem))
            copies.append(pltpu.async_copy(
                tflat_hbm.at[idx_i.at[pl.ds(j * CH, CH)]],
                dst_i.at[pl.ds(j * CH, CH)], sem))
        return copies

    def dot_q(q, dst_u, dst_i):
        h0 = q * QROWS

        def dot(g, carry):
            acc = jnp.zeros((16,), jnp.float32)
            for k in range(EMBED_DIM):
                p = g * 1024 + k * 16
                acc = acc + dst_u[pl.ds(p, 16)] * dst_i[pl.ds(p, 16)]
            out_v[pl.ds(h0 + g * 16, 16)] = acc
            return carry

        lax.fori_loop(0, QG, dot, 0)

    nslot = len(bufs)
    inflight = {}
    for q in range(NQ):
        slot = q % nslot
        if q >= nslot:
            for cp in inflight[q - nslot]:
                cp.wait()
            dot_q(q - nslot, bufs[slot][2], bufs[slot][3])
        gen_q(q, bufs[slot][0], bufs[slot][1])
        inflight[q] = fire_q(*bufs[slot])
    for q in range(max(0, NQ - nslot), NQ):
        slot = q % nslot
        for cp in inflight[q]:
            cp.wait()
        dot_q(q, bufs[slot][2], bufs[slot][3])

    pltpu.sync_copy(out_v, out_hbm.at[pl.ds(base, BPW)])


def _sc_dot(xu, xi, tflat):
    mesh = plsc.VectorSubcoreMesh(core_axis_name="c", subcore_axis_name="s")
    kern = functools.partial(
        pl.kernel,
        out_type=jax.ShapeDtypeStruct((BATCH,), jnp.float32),
        mesh=mesh,
        compiler_params=pltpu.CompilerParams(needs_layout_passes=False,
                                             use_tc_tiling_on_sc=False),
        scratch_types=[
            pltpu.VMEM((BPW,), jnp.int32),
            pltpu.VMEM((BPW,), jnp.int32),
            pltpu.VMEM((IDX_PER_Q,), jnp.int32),
            pltpu.VMEM((IDX_PER_Q,), jnp.int32),
            pltpu.VMEM((IDX_PER_Q,), jnp.int32),
            pltpu.VMEM((IDX_PER_Q,), jnp.int32),
            pltpu.VMEM((IDX_PER_Q,), jnp.int32),
            pltpu.VMEM((IDX_PER_Q,), jnp.int32),
            pltpu.VMEM((IDX_PER_Q,), jnp.float32),
            pltpu.VMEM((IDX_PER_Q,), jnp.float32),
            pltpu.VMEM((IDX_PER_Q,), jnp.float32),
            pltpu.VMEM((IDX_PER_Q,), jnp.float32),
            pltpu.VMEM((IDX_PER_Q,), jnp.float32),
            pltpu.VMEM((IDX_PER_Q,), jnp.float32),
            pltpu.VMEM((BPW,), jnp.float32),
            pltpu.SemaphoreType.DMA,
            pltpu.SemaphoreType.DMA,
            pltpu.SemaphoreType.DMA,
            pltpu.SemaphoreType.DMA,
        ],
    )(_sc_body)
    return kern(xu, xi, tflat)


def kernel(x, table):
    x = x.astype(jnp.int32)
    tflat = (table.T.reshape(8, 8, ROWS // 128, 128)
             .transpose(0, 2, 1, 3).reshape(-1))
    y = _sc_dot(x[:, 0], x[:, 1], tflat)
    return y.reshape(BATCH, 1)
